# Initial kernel scaffold; baseline (speedup 1.0000x reference)
import jax
import jax.numpy as jnp
from jax.experimental import pallas as pl


def kernel(x, edge_index, W1l, b1, W1r, W2l, b2, W2r, W3, b3):
    raise NotImplementedError("write your pallas kernel here")


# speedup vs baseline: 2.3463x; 2.3463x over previous
import functools

import jax
import jax.numpy as jnp
from jax import lax
from jax.experimental import pallas as pl
from jax.experimental.pallas import tpu as pltpu
from jax.experimental.pallas import tpu_sc as plsc

N = 10000
E = 320000
D = 128

NC = 2
NS = 16
NW = NC * NS
C = 128
CH = 80
EPAD = NW * C * CH
NP = 10240
RPT = NP // NS
LC = 16
IR = 2 * CH
IRPAD = 4 * CH
NG = 4
GCH = CH // NG
GR = 2 * GCH


def _sc_body(table, idxs, z128, acc_out,
             idx_v, buf, acc_sh, sem):
    ci = lax.axis_index("c")
    si = lax.axis_index("s")
    w = ci * NS + si
    r0 = si * RPT
    pltpu.sync_copy(z128, buf)
    for k in range(RPT // C):
        pltpu.async_copy(buf, acc_sh.at[pl.ds(r0 + k * C, C)], sem).wait()
    plsc.subcore_barrier()

    def group(g, carry):
        pltpu.sync_copy(idxs.at[w, pl.ds(g * GR, GR)], idx_v)

        def body(t, carry2):
            pltpu.async_copy(table.at[idx_v.at[2 * t]], buf, sem).wait()
            pltpu.async_copy(buf, acc_sh.at[idx_v.at[2 * t + 1]], sem,
                             add=True).wait()
            return carry2

        lax.fori_loop(0, GCH, body, 0)
        return carry

    lax.fori_loop(0, NG, group, 0)
    plsc.subcore_barrier()
    for k in range(RPT // C):
        pltpu.async_copy(acc_sh.at[pl.ds(r0 + k * C, C)], buf, sem).wait()
        pltpu.sync_copy(buf, acc_out.at[ci, pl.ds(r0 + k * C, C)])


def _sc_aggregate(table, idxs, z128):
    mesh = plsc.VectorSubcoreMesh(core_axis_name="c", subcore_axis_name="s")
    kern = pl.kernel(
        _sc_body,
        mesh=mesh,
        out_type=jax.ShapeDtypeStruct((NC, NP, D), jnp.float32),
        scratch_types=[
            pltpu.VMEM((GR, C), jnp.int32),
            pltpu.VMEM((C, D), jnp.float32),
            pltpu.VMEM_SHARED((NP, D), jnp.float32),
            pltpu.SemaphoreType.DMA,
        ],
    )
    return kern(table, idxs, z128)


_DOT = (((1,), (1,)), ((), ()))


def _pre_body(x_ref, wl_ref, wr_ref, t_ref, r_ref):
    xb = x_ref[...]
    t_ref[...] = lax.dot_general(xb, wl_ref[...], _DOT,
                                 preferred_element_type=jnp.float32)
    r_ref[...] = lax.dot_general(xb, wr_ref[...], _DOT,
                                 preferred_element_type=jnp.float32)


def _tc_pre(x, W1l, W1r):
    bm = 1000
    return pl.pallas_call(
        _pre_body,
        grid=(N // bm,),
        in_specs=[pl.BlockSpec((bm, D), lambda i: (i, 0)),
                  pl.BlockSpec((D, D), lambda i: (0, 0)),
                  pl.BlockSpec((D, D), lambda i: (0, 0))],
        out_specs=[pl.BlockSpec((bm, D), lambda i: (i, 0)),
                   pl.BlockSpec((bm, D), lambda i: (i, 0))],
        out_shape=[jax.ShapeDtypeStruct((N, D), jnp.float32),
                   jax.ShapeDtypeStruct((N, D), jnp.float32)],
    )(x, W1l, W1r)


def _mean_relu(acc_ref, cnt_ref, r_ref, b_ref):
    acc = acc_ref[0] + acc_ref[1]
    cnt = cnt_ref[0][:, 0:1] + cnt_ref[1][:, 0:1]
    mean = acc / jnp.maximum(cnt, 1.0)
    return jnp.maximum(mean + b_ref[...] + r_ref[...], 0.0)


def _mid_body(acc_ref, cnt_ref, r_ref, b_ref, wl_ref, wr_ref, t_ref, rn_ref):
    h = _mean_relu(acc_ref, cnt_ref, r_ref, b_ref)
    t_ref[...] = lax.dot_general(h, wl_ref[...], _DOT,
                                 preferred_element_type=jnp.float32)
    rn_ref[...] = lax.dot_general(h, wr_ref[...], _DOT,
                                  preferred_element_type=jnp.float32)


def _tc_mid(acc, cnt, r, b, Wl, Wr):
    bm = 1000
    return pl.pallas_call(
        _mid_body,
        grid=(N // bm,),
        in_specs=[pl.BlockSpec((NC, bm, D), lambda i: (0, i, 0)),
                  pl.BlockSpec((NC, bm, D), lambda i: (0, i, 0)),
                  pl.BlockSpec((bm, D), lambda i: (i, 0)),
                  pl.BlockSpec((1, D), lambda i: (0, 0)),
                  pl.BlockSpec((D, D), lambda i: (0, 0)),
                  pl.BlockSpec((D, D), lambda i: (0, 0))],
        out_specs=[pl.BlockSpec((bm, D), lambda i: (i, 0)),
                   pl.BlockSpec((bm, D), lambda i: (i, 0))],
        out_shape=[jax.ShapeDtypeStruct((N, D), jnp.float32),
                   jax.ShapeDtypeStruct((N, D), jnp.float32)],
    )(acc, cnt, r, b.reshape(1, D), Wl, Wr)


def _fin_body(acc_ref, cnt_ref, r_ref, b_ref, w3_ref, b3_ref, o_ref):
    h = _mean_relu(acc_ref, cnt_ref, r_ref, b_ref)
    o_ref[...] = lax.dot_general(h, w3_ref[...], _DOT,
                                 preferred_element_type=jnp.float32) + b3_ref[...]


def _tc_fin(acc, cnt, r, b, W3, b3):
    bm = 1000
    return pl.pallas_call(
        _fin_body,
        grid=(N // bm,),
        in_specs=[pl.BlockSpec((NC, bm, D), lambda i: (0, i, 0)),
                  pl.BlockSpec((NC, bm, D), lambda i: (0, i, 0)),
                  pl.BlockSpec((bm, D), lambda i: (i, 0)),
                  pl.BlockSpec((1, D), lambda i: (0, 0)),
                  pl.BlockSpec((D, D), lambda i: (0, 0)),
                  pl.BlockSpec((1, D), lambda i: (0, 0))],
        out_specs=pl.BlockSpec((bm, D), lambda i: (i, 0)),
        out_shape=jax.ShapeDtypeStruct((N, D), jnp.float32),
    )(acc, cnt, r, b.reshape(1, D), W3, b3.reshape(1, D))


def kernel(x, edge_index, W1l, b1, W1r, W2l, b2, W2r, W3, b3):
    src = edge_index[0]
    dst = edge_index[1]
    pad = EPAD - E
    src_p = jnp.concatenate([src, jnp.zeros((pad,), jnp.int32)]).reshape(NW, CH, C)
    dst_p = jnp.concatenate([dst, jnp.full((pad,), N, jnp.int32)]).reshape(NW, CH, C)
    inter = jnp.stack([src_p, dst_p], axis=2).reshape(NW, IR, C)
    dead = jnp.zeros((NW, IRPAD - IR, C), jnp.int32)
    idxs = jnp.concatenate([inter, dead], axis=1)
    z128 = jnp.zeros((C, D), jnp.float32)
    ones_t = jnp.ones((N, D), jnp.float32)

    cnt = _sc_aggregate(ones_t, idxs, z128)
    t1, r1 = _tc_pre(x, W1l, W1r)
    acc1 = _sc_aggregate(t1, idxs, z128)
    t2, r2 = _tc_mid(acc1, cnt, r1, b1, W2l, W2r)
    acc2 = _sc_aggregate(t2, idxs, z128)
    return _tc_fin(acc2, cnt, r2, b2, W3, b3)


# speedup vs baseline: 2.6004x; 1.1083x over previous
import functools

import jax
import jax.numpy as jnp
from jax import lax
from jax.experimental import pallas as pl
from jax.experimental.pallas import tpu as pltpu
from jax.experimental.pallas import tpu_sc as plsc

N = 10000
E = 320000
D = 128

NC = 2
NS = 16
NW = NC * NS
C = 128
CH = 80
EPAD = NW * C * CH
NP = 10240
RPT = NP // NS
LC = 16
IR = 2 * CH
IRPAD = 4 * CH
NG = 2
GCH = CH // NG
GR = 2 * GCH


def _sc_body(table, idxs, z128, acc_out,
             idx_v, buf_a, buf_b, acc_sh, sem_a, sem_b, sem_s):
    ci = lax.axis_index("c")
    si = lax.axis_index("s")
    w = ci * NS + si
    r0 = si * RPT
    pltpu.sync_copy(z128, buf_a)
    for k in range(RPT // C):
        pltpu.async_copy(buf_a, acc_sh.at[pl.ds(r0 + k * C, C)], sem_a).wait()
    plsc.subcore_barrier()

    def group(g, carry):
        pltpu.sync_copy(idxs.at[w, pl.ds(g * GR, GR)], idx_v)
        pltpu.async_copy(table.at[idx_v.at[0]], buf_a, sem_a)

        def body(tt, carry2):
            j = 2 * tt
            pltpu.make_async_copy(table.at[idx_v.at[2 * j]], buf_a,
                                  sem_a).wait()
            pltpu.async_copy(table.at[idx_v.at[2 * j + 2]], buf_b, sem_b)
            pltpu.async_copy(buf_a, acc_sh.at[idx_v.at[2 * j + 1]], sem_s,
                             add=True).wait()

            @pl.when(j + 2 < GCH)
            def _():
                pltpu.async_copy(table.at[idx_v.at[2 * j + 4]], buf_a, sem_a)

            pltpu.make_async_copy(table.at[idx_v.at[2 * j + 2]], buf_b,
                                  sem_b).wait()
            pltpu.async_copy(buf_b, acc_sh.at[idx_v.at[2 * j + 3]], sem_s,
                             add=True).wait()
            return carry2

        lax.fori_loop(0, GCH // 2, body, 0)
        return carry

    lax.fori_loop(0, NG, group, 0)
    plsc.subcore_barrier()
    for k in range(RPT // C):
        pltpu.async_copy(acc_sh.at[pl.ds(r0 + k * C, C)], buf_a, sem_a).wait()
        pltpu.sync_copy(buf_a, acc_out.at[ci, pl.ds(r0 + k * C, C)])


def _sc_aggregate(table, idxs, z128):
    mesh = plsc.VectorSubcoreMesh(core_axis_name="c", subcore_axis_name="s")
    kern = pl.kernel(
        _sc_body,
        mesh=mesh,
        out_type=jax.ShapeDtypeStruct((NC, NP, D), jnp.float32),
        scratch_types=[
            pltpu.VMEM((GR, C), jnp.int32),
            pltpu.VMEM((C, D), jnp.float32),
            pltpu.VMEM((C, D), jnp.float32),
            pltpu.VMEM_SHARED((NP, D), jnp.float32),
            pltpu.SemaphoreType.DMA,
            pltpu.SemaphoreType.DMA,
            pltpu.SemaphoreType.DMA,
        ],
    )
    return kern(table, idxs, z128)


_DOT = (((1,), (1,)), ((), ()))


def _pre_body(x_ref, wl_ref, wr_ref, t_ref, r_ref):
    xb = x_ref[...]
    t_ref[...] = lax.dot_general(xb, wl_ref[...], _DOT,
                                 preferred_element_type=jnp.float32)
    r_ref[...] = lax.dot_general(xb, wr_ref[...], _DOT,
                                 preferred_element_type=jnp.float32)


def _tc_pre(x, W1l, W1r):
    bm = 1000
    return pl.pallas_call(
        _pre_body,
        grid=(N // bm,),
        in_specs=[pl.BlockSpec((bm, D), lambda i: (i, 0)),
                  pl.BlockSpec((D, D), lambda i: (0, 0)),
                  pl.BlockSpec((D, D), lambda i: (0, 0))],
        out_specs=[pl.BlockSpec((bm, D), lambda i: (i, 0)),
                   pl.BlockSpec((bm, D), lambda i: (i, 0))],
        out_shape=[jax.ShapeDtypeStruct((N, D), jnp.float32),
                   jax.ShapeDtypeStruct((N, D), jnp.float32)],
    )(x, W1l, W1r)


def _mean_relu(acc_ref, cnt_ref, r_ref, b_ref):
    acc = acc_ref[0] + acc_ref[1]
    cnt = cnt_ref[0][:, 0:1] + cnt_ref[1][:, 0:1]
    mean = acc / jnp.maximum(cnt, 1.0)
    return jnp.maximum(mean + b_ref[...] + r_ref[...], 0.0)


def _mid_body(acc_ref, cnt_ref, r_ref, b_ref, wl_ref, wr_ref, t_ref, rn_ref):
    h = _mean_relu(acc_ref, cnt_ref, r_ref, b_ref)
    t_ref[...] = lax.dot_general(h, wl_ref[...], _DOT,
                                 preferred_element_type=jnp.float32)
    rn_ref[...] = lax.dot_general(h, wr_ref[...], _DOT,
                                  preferred_element_type=jnp.float32)


def _tc_mid(acc, cnt, r, b, Wl, Wr):
    bm = 1000
    return pl.pallas_call(
        _mid_body,
        grid=(N // bm,),
        in_specs=[pl.BlockSpec((NC, bm, D), lambda i: (0, i, 0)),
                  pl.BlockSpec((NC, bm, D), lambda i: (0, i, 0)),
                  pl.BlockSpec((bm, D), lambda i: (i, 0)),
                  pl.BlockSpec((1, D), lambda i: (0, 0)),
                  pl.BlockSpec((D, D), lambda i: (0, 0)),
                  pl.BlockSpec((D, D), lambda i: (0, 0))],
        out_specs=[pl.BlockSpec((bm, D), lambda i: (i, 0)),
                   pl.BlockSpec((bm, D), lambda i: (i, 0))],
        out_shape=[jax.ShapeDtypeStruct((N, D), jnp.float32),
                   jax.ShapeDtypeStruct((N, D), jnp.float32)],
    )(acc, cnt, r, b.reshape(1, D), Wl, Wr)


def _fin_body(acc_ref, cnt_ref, r_ref, b_ref, w3_ref, b3_ref, o_ref):
    h = _mean_relu(acc_ref, cnt_ref, r_ref, b_ref)
    o_ref[...] = lax.dot_general(h, w3_ref[...], _DOT,
                                 preferred_element_type=jnp.float32) + b3_ref[...]


def _tc_fin(acc, cnt, r, b, W3, b3):
    bm = 1000
    return pl.pallas_call(
        _fin_body,
        grid=(N // bm,),
        in_specs=[pl.BlockSpec((NC, bm, D), lambda i: (0, i, 0)),
                  pl.BlockSpec((NC, bm, D), lambda i: (0, i, 0)),
                  pl.BlockSpec((bm, D), lambda i: (i, 0)),
                  pl.BlockSpec((1, D), lambda i: (0, 0)),
                  pl.BlockSpec((D, D), lambda i: (0, 0)),
                  pl.BlockSpec((1, D), lambda i: (0, 0))],
        out_specs=pl.BlockSpec((bm, D), lambda i: (i, 0)),
        out_shape=jax.ShapeDtypeStruct((N, D), jnp.float32),
    )(acc, cnt, r, b.reshape(1, D), W3, b3.reshape(1, D))


def kernel(x, edge_index, W1l, b1, W1r, W2l, b2, W2r, W3, b3):
    src = edge_index[0]
    dst = edge_index[1]
    pad = EPAD - E
    src_p = jnp.concatenate([src, jnp.zeros((pad,), jnp.int32)]).reshape(NW, CH, C)
    dst_p = jnp.concatenate([dst, jnp.full((pad,), N, jnp.int32)]).reshape(NW, CH, C)
    inter = jnp.stack([src_p, dst_p], axis=2).reshape(NW, IR, C)
    dead = jnp.zeros((NW, IRPAD - IR, C), jnp.int32)
    idxs = jnp.concatenate([inter, dead], axis=1)
    z128 = jnp.zeros((C, D), jnp.float32)
    ones_t = jnp.ones((N, D), jnp.float32)

    cnt = _sc_aggregate(ones_t, idxs, z128)
    t1, r1 = _tc_pre(x, W1l, W1r)
    acc1 = _sc_aggregate(t1, idxs, z128)
    t2, r2 = _tc_mid(acc1, cnt, r1, b1, W2l, W2r)
    acc2 = _sc_aggregate(t2, idxs, z128)
    return _tc_fin(acc2, cnt, r2, b2, W3, b3)


# speedup vs baseline: 3.5256x; 1.3558x over previous
import functools

import jax
import jax.numpy as jnp
from jax import lax
from jax.experimental import pallas as pl
from jax.experimental.pallas import tpu as pltpu
from jax.experimental.pallas import tpu_sc as plsc

N = 10000
E = 320000
D = 128

NC = 2
NS = 16
NW = NC * NS
C = 128
CH = 80
EPAD = NW * C * CH
NP = 10240
RPT = NP // NS
LC = 16
IR = 2 * CH
IRPAD = 4 * CH
NG = 2
GCH = CH // NG
GR = 2 * GCH


def _sc_body(table, idxs, z128, acc_out,
             idx_v, buf_a, buf_b, acc_sh, sem_a, sem_b, sem_s):
    ci = lax.axis_index("c")
    si = lax.axis_index("s")
    w = ci * NS + si
    r0 = si * RPT
    pltpu.sync_copy(z128, buf_a)
    for k in range(RPT // C):
        pltpu.async_copy(buf_a, acc_sh.at[pl.ds(r0 + k * C, C)], sem_a).wait()
    plsc.subcore_barrier()

    def group(g, carry):
        pltpu.sync_copy(idxs.at[w, pl.ds(g * GR, GR)], idx_v)
        pltpu.async_copy(table.at[idx_v.at[0]], buf_a, sem_a)

        def body(tt, carry2):
            j = 2 * tt
            pltpu.make_async_copy(table.at[idx_v.at[2 * j]], buf_a,
                                  sem_a).wait()
            pltpu.async_copy(table.at[idx_v.at[2 * j + 2]], buf_b, sem_b)
            pltpu.async_copy(buf_a, acc_sh.at[idx_v.at[2 * j + 1]], sem_s,
                             add=True).wait()

            @pl.when(j + 2 < GCH)
            def _():
                pltpu.async_copy(table.at[idx_v.at[2 * j + 4]], buf_a, sem_a)

            pltpu.make_async_copy(table.at[idx_v.at[2 * j + 2]], buf_b,
                                  sem_b).wait()
            pltpu.async_copy(buf_b, acc_sh.at[idx_v.at[2 * j + 3]], sem_s,
                             add=True).wait()
            return carry2

        lax.fori_loop(0, GCH // 2, body, 0)
        return carry

    lax.fori_loop(0, NG, group, 0)
    plsc.subcore_barrier()
    for k in range(RPT // C):
        pltpu.async_copy(acc_sh.at[pl.ds(r0 + k * C, C)], buf_a, sem_a).wait()
        pltpu.sync_copy(buf_a, acc_out.at[ci, pl.ds(r0 + k * C, C)])


def _sc_aggregate(table, idxs, z128):
    mesh = plsc.VectorSubcoreMesh(core_axis_name="c", subcore_axis_name="s")
    kern = pl.kernel(
        _sc_body,
        mesh=mesh,
        out_type=jax.ShapeDtypeStruct((NC, NP, D), jnp.float32),
        scratch_types=[
            pltpu.VMEM((GR, C), jnp.int32),
            pltpu.VMEM((C, D), jnp.float32),
            pltpu.VMEM((C, D), jnp.float32),
            pltpu.VMEM_SHARED((NP, D), jnp.float32),
            pltpu.SemaphoreType.DMA,
            pltpu.SemaphoreType.DMA,
            pltpu.SemaphoreType.DMA,
        ],
    )
    return kern(table, idxs, z128)


def _sc_count_body(idxs, z128, ones_in, cnt_out,
                   idx_v, zbuf, obuf, cnt_sh, sem_a, sem_s):
    ci = lax.axis_index("c")
    si = lax.axis_index("s")
    w = ci * NS + si
    r0 = si * RPT
    pltpu.sync_copy(z128, zbuf)
    for k in range(RPT // C):
        pltpu.async_copy(zbuf, cnt_sh.at[pl.ds(r0 + k * C, C)], sem_a).wait()
    pltpu.sync_copy(ones_in, obuf)
    plsc.subcore_barrier()

    def group(g, carry):
        pltpu.sync_copy(idxs.at[w, pl.ds(g * GR, GR)], idx_v)

        def body(t, carry2):
            pltpu.async_copy(obuf, cnt_sh.at[idx_v.at[2 * t + 1]], sem_s,
                             add=True).wait()
            return carry2

        lax.fori_loop(0, GCH, body, 0)
        return carry

    lax.fori_loop(0, NG, group, 0)
    plsc.subcore_barrier()
    for k in range(RPT // C):
        pltpu.async_copy(cnt_sh.at[pl.ds(r0 + k * C, C)], zbuf, sem_a).wait()
        pltpu.sync_copy(zbuf, cnt_out.at[ci, pl.ds(r0 + k * C, C)])


def _sc_count(idxs, z128, ones_in):
    mesh = plsc.VectorSubcoreMesh(core_axis_name="c", subcore_axis_name="s")
    kern = pl.kernel(
        _sc_count_body,
        mesh=mesh,
        out_type=jax.ShapeDtypeStruct((NC, NP, D), jnp.float32),
        scratch_types=[
            pltpu.VMEM((GR, C), jnp.int32),
            pltpu.VMEM((C, D), jnp.float32),
            pltpu.VMEM((C, D), jnp.float32),
            pltpu.VMEM_SHARED((NP, D), jnp.float32),
            pltpu.SemaphoreType.DMA,
            pltpu.SemaphoreType.DMA,
        ],
    )
    return kern(idxs, z128, ones_in)


_DOT = (((1,), (1,)), ((), ()))


def _pre_body(x_ref, wl_ref, wr_ref, t_ref, r_ref):
    xb = x_ref[...]
    t_ref[...] = lax.dot_general(xb, wl_ref[...], _DOT,
                                 preferred_element_type=jnp.float32)
    r_ref[...] = lax.dot_general(xb, wr_ref[...], _DOT,
                                 preferred_element_type=jnp.float32)


def _tc_pre(x, W1l, W1r):
    bm = 1000
    return pl.pallas_call(
        _pre_body,
        grid=(N // bm,),
        in_specs=[pl.BlockSpec((bm, D), lambda i: (i, 0)),
                  pl.BlockSpec((D, D), lambda i: (0, 0)),
                  pl.BlockSpec((D, D), lambda i: (0, 0))],
        out_specs=[pl.BlockSpec((bm, D), lambda i: (i, 0)),
                   pl.BlockSpec((bm, D), lambda i: (i, 0))],
        out_shape=[jax.ShapeDtypeStruct((N, D), jnp.float32),
                   jax.ShapeDtypeStruct((N, D), jnp.float32)],
    )(x, W1l, W1r)


def _mean_relu(acc_ref, cnt_ref, r_ref, b_ref):
    acc = acc_ref[0] + acc_ref[1]
    cnt = cnt_ref[0][:, 0:1] + cnt_ref[1][:, 0:1]
    mean = acc / jnp.maximum(cnt, 1.0)
    return jnp.maximum(mean + b_ref[...] + r_ref[...], 0.0)


def _mid_body(acc_ref, cnt_ref, r_ref, b_ref, wl_ref, wr_ref, t_ref, rn_ref):
    h = _mean_relu(acc_ref, cnt_ref, r_ref, b_ref)
    t_ref[...] = lax.dot_general(h, wl_ref[...], _DOT,
                                 preferred_element_type=jnp.float32)
    rn_ref[...] = lax.dot_general(h, wr_ref[...], _DOT,
                                  preferred_element_type=jnp.float32)


def _tc_mid(acc, cnt, r, b, Wl, Wr):
    bm = 1000
    return pl.pallas_call(
        _mid_body,
        grid=(N // bm,),
        in_specs=[pl.BlockSpec((NC, bm, D), lambda i: (0, i, 0)),
                  pl.BlockSpec((NC, bm, D), lambda i: (0, i, 0)),
                  pl.BlockSpec((bm, D), lambda i: (i, 0)),
                  pl.BlockSpec((1, D), lambda i: (0, 0)),
                  pl.BlockSpec((D, D), lambda i: (0, 0)),
                  pl.BlockSpec((D, D), lambda i: (0, 0))],
        out_specs=[pl.BlockSpec((bm, D), lambda i: (i, 0)),
                   pl.BlockSpec((bm, D), lambda i: (i, 0))],
        out_shape=[jax.ShapeDtypeStruct((N, D), jnp.float32),
                   jax.ShapeDtypeStruct((N, D), jnp.float32)],
    )(acc, cnt, r, b.reshape(1, D), Wl, Wr)


def _fin_body(acc_ref, cnt_ref, r_ref, b_ref, w3_ref, b3_ref, o_ref):
    h = _mean_relu(acc_ref, cnt_ref, r_ref, b_ref)
    o_ref[...] = lax.dot_general(h, w3_ref[...], _DOT,
                                 preferred_element_type=jnp.float32) + b3_ref[...]


def _tc_fin(acc, cnt, r, b, W3, b3):
    bm = 1000
    return pl.pallas_call(
        _fin_body,
        grid=(N // bm,),
        in_specs=[pl.BlockSpec((NC, bm, D), lambda i: (0, i, 0)),
                  pl.BlockSpec((NC, bm, D), lambda i: (0, i, 0)),
                  pl.BlockSpec((bm, D), lambda i: (i, 0)),
                  pl.BlockSpec((1, D), lambda i: (0, 0)),
                  pl.BlockSpec((D, D), lambda i: (0, 0)),
                  pl.BlockSpec((1, D), lambda i: (0, 0))],
        out_specs=pl.BlockSpec((bm, D), lambda i: (i, 0)),
        out_shape=jax.ShapeDtypeStruct((N, D), jnp.float32),
    )(acc, cnt, r, b.reshape(1, D), W3, b3.reshape(1, D))


def kernel(x, edge_index, W1l, b1, W1r, W2l, b2, W2r, W3, b3):
    src = edge_index[0]
    dst = edge_index[1]
    pad = EPAD - E
    src_p = jnp.concatenate([src, jnp.zeros((pad,), jnp.int32)]).reshape(NW, CH, C)
    dst_p = jnp.concatenate([dst, jnp.full((pad,), N, jnp.int32)]).reshape(NW, CH, C)
    inter = jnp.stack([src_p, dst_p], axis=2).reshape(NW, IR, C)
    dead = jnp.zeros((NW, IRPAD - IR, C), jnp.int32)
    idxs = jnp.concatenate([inter, dead], axis=1)
    z128 = jnp.zeros((C, D), jnp.float32)
    ones_in = jnp.ones((C, D), jnp.float32)

    cnt = _sc_count(idxs, z128, ones_in)
    t1, r1 = _tc_pre(x, W1l, W1r)
    acc1 = _sc_aggregate(t1, idxs, z128)
    t2, r2 = _tc_mid(acc1, cnt, r1, b1, W2l, W2r)
    acc2 = _sc_aggregate(t2, idxs, z128)
    return _tc_fin(acc2, cnt, r2, b2, W3, b3)


# speedup vs baseline: 3.5493x; 1.0067x over previous
import functools

import jax
import jax.numpy as jnp
from jax import lax
from jax.experimental import pallas as pl
from jax.experimental.pallas import tpu as pltpu
from jax.experimental.pallas import tpu_sc as plsc

N = 10000
E = 320000
D = 128

NC = 2
NS = 16
NW = NC * NS
C = 128
CH = 80
EPAD = NW * C * CH
NP = 10240
RPT = NP // NS
LC = 16
IR = 2 * CH
IRPAD = 4 * CH
NG = 2
GCH = CH // NG
GR = 2 * GCH


def _zero_acc(z128, buf_a, acc_sh, r0, sem_a):
    pltpu.sync_copy(z128, buf_a)
    for k in range(RPT // C):
        pltpu.async_copy(buf_a, acc_sh.at[pl.ds(r0 + k * C, C)], sem_a).wait()


def _drain_acc(acc_sh, out, ci, r0, buf_a, sem_a):
    for k in range(RPT // C):
        pltpu.async_copy(acc_sh.at[pl.ds(r0 + k * C, C)], buf_a, sem_a).wait()
        pltpu.sync_copy(buf_a, out.at[ci, pl.ds(r0 + k * C, C)])


def _sc_body_fused(table, idxs, z128, ones_in, acc_out, cnt_out,
                   idx_v, buf_a, buf_b, acc_sh, sem_a, sem_b, sem_s):
    _sc_agg_impl(table, idxs, z128, acc_out, idx_v, buf_a, buf_b, acc_sh,
                 sem_a, sem_b, sem_s)
    ci = lax.axis_index("c")
    si = lax.axis_index("s")
    w = ci * NS + si
    r0 = si * RPT
    plsc.subcore_barrier()
    _zero_acc(z128, buf_a, acc_sh, r0, sem_a)
    pltpu.sync_copy(ones_in, buf_b)
    plsc.subcore_barrier()

    def cgroup(g, carry):
        pltpu.sync_copy(idxs.at[w, pl.ds(g * GR, GR)], idx_v)

        def cbody(t, carry2):
            pltpu.async_copy(buf_b, acc_sh.at[idx_v.at[2 * t + 1]], sem_s,
                             add=True).wait()
            return carry2

        lax.fori_loop(0, GCH, cbody, 0)
        return carry

    lax.fori_loop(0, NG, cgroup, 0)
    plsc.subcore_barrier()
    _drain_acc(acc_sh, cnt_out, ci, r0, buf_a, sem_a)


def _sc_body(table, idxs, z128, acc_out,
             idx_v, buf_a, buf_b, acc_sh, sem_a, sem_b, sem_s):
    _sc_agg_impl(table, idxs, z128, acc_out, idx_v, buf_a, buf_b, acc_sh,
                 sem_a, sem_b, sem_s)


def _sc_agg_impl(table, idxs, z128, acc_out,
                 idx_v, buf_a, buf_b, acc_sh, sem_a, sem_b, sem_s):
    ci = lax.axis_index("c")
    si = lax.axis_index("s")
    w = ci * NS + si
    r0 = si * RPT
    _zero_acc(z128, buf_a, acc_sh, r0, sem_a)
    plsc.subcore_barrier()

    HC = C // 2

    def gat(j, buf, sem):
        pltpu.async_copy(table.at[idx_v.at[2 * j, pl.ds(0, HC)]],
                         buf.at[pl.ds(0, HC)], sem)
        pltpu.async_copy(table.at[idx_v.at[2 * j, pl.ds(HC, HC)]],
                         buf.at[pl.ds(HC, HC)], sem)

    def wat(j, buf, sem):
        pltpu.make_async_copy(table.at[idx_v.at[2 * j, pl.ds(0, HC)]],
                              buf.at[pl.ds(0, HC)], sem).wait()
        pltpu.make_async_copy(table.at[idx_v.at[2 * j, pl.ds(HC, HC)]],
                              buf.at[pl.ds(HC, HC)], sem).wait()

    def group(g, carry):
        pltpu.sync_copy(idxs.at[w, pl.ds(g * GR, GR)], idx_v)
        gat(0, buf_a, sem_a)
        gat(1, buf_b, sem_b)

        def body(tt, carry2):
            j = 2 * tt
            wat(j, buf_a, sem_a)
            pltpu.async_copy(buf_a, acc_sh.at[idx_v.at[2 * j + 1]], sem_s,
                             add=True).wait()

            @pl.when(j + 2 < GCH)
            def _():
                gat(j + 2, buf_a, sem_a)

            wat(j + 1, buf_b, sem_b)
            pltpu.async_copy(buf_b, acc_sh.at[idx_v.at[2 * j + 3]], sem_s,
                             add=True).wait()

            @pl.when(j + 3 < GCH)
            def _():
                gat(j + 3, buf_b, sem_b)

            return carry2

        lax.fori_loop(0, GCH // 2, body, 0)
        return carry

    lax.fori_loop(0, NG, group, 0)
    plsc.subcore_barrier()
    _drain_acc(acc_sh, acc_out, ci, r0, buf_a, sem_a)


def _sc_aggregate(table, idxs, z128):
    mesh = plsc.VectorSubcoreMesh(core_axis_name="c", subcore_axis_name="s")
    kern = pl.kernel(
        _sc_body,
        mesh=mesh,
        out_type=jax.ShapeDtypeStruct((NC, NP, D), jnp.float32),
        scratch_types=[
            pltpu.VMEM((GR, C), jnp.int32),
            pltpu.VMEM((C, D), jnp.float32),
            pltpu.VMEM((C, D), jnp.float32),
            pltpu.VMEM_SHARED((NP, D), jnp.float32),
            pltpu.SemaphoreType.DMA,
            pltpu.SemaphoreType.DMA,
            pltpu.SemaphoreType.DMA,
        ],
    )
    return kern(table, idxs, z128)


def _sc_aggregate_cnt(table, idxs, z128, ones_in):
    mesh = plsc.VectorSubcoreMesh(core_axis_name="c", subcore_axis_name="s")
    kern = pl.kernel(
        _sc_body_fused,
        mesh=mesh,
        out_type=(jax.ShapeDtypeStruct((NC, NP, D), jnp.float32),
                  jax.ShapeDtypeStruct((NC, NP, D), jnp.float32)),
        scratch_types=[
            pltpu.VMEM((GR, C), jnp.int32),
            pltpu.VMEM((C, D), jnp.float32),
            pltpu.VMEM((C, D), jnp.float32),
            pltpu.VMEM_SHARED((NP, D), jnp.float32),
            pltpu.SemaphoreType.DMA,
            pltpu.SemaphoreType.DMA,
            pltpu.SemaphoreType.DMA,
        ],
    )
    return kern(table, idxs, z128, ones_in)


_DOT = (((1,), (1,)), ((), ()))


def _pre_body(x_ref, wl_ref, wr_ref, t_ref, r_ref):
    xb = x_ref[...]
    t_ref[...] = lax.dot_general(xb, wl_ref[...], _DOT,
                                 preferred_element_type=jnp.float32)
    r_ref[...] = lax.dot_general(xb, wr_ref[...], _DOT,
                                 preferred_element_type=jnp.float32)


def _tc_pre(x, W1l, W1r):
    bm = 1000
    return pl.pallas_call(
        _pre_body,
        grid=(N // bm,),
        in_specs=[pl.BlockSpec((bm, D), lambda i: (i, 0)),
                  pl.BlockSpec((D, D), lambda i: (0, 0)),
                  pl.BlockSpec((D, D), lambda i: (0, 0))],
        out_specs=[pl.BlockSpec((bm, D), lambda i: (i, 0)),
                   pl.BlockSpec((bm, D), lambda i: (i, 0))],
        out_shape=[jax.ShapeDtypeStruct((N, D), jnp.float32),
                   jax.ShapeDtypeStruct((N, D), jnp.float32)],
    )(x, W1l, W1r)


def _mean_relu(acc_ref, cnt_ref, r_ref, b_ref):
    acc = acc_ref[0] + acc_ref[1]
    cnt = cnt_ref[0][:, 0:1] + cnt_ref[1][:, 0:1]
    mean = acc / jnp.maximum(cnt, 1.0)
    return jnp.maximum(mean + b_ref[...] + r_ref[...], 0.0)


def _mid_body(acc_ref, cnt_ref, r_ref, b_ref, wl_ref, wr_ref, t_ref, rn_ref):
    h = _mean_relu(acc_ref, cnt_ref, r_ref, b_ref)
    t_ref[...] = lax.dot_general(h, wl_ref[...], _DOT,
                                 preferred_element_type=jnp.float32)
    rn_ref[...] = lax.dot_general(h, wr_ref[...], _DOT,
                                  preferred_element_type=jnp.float32)


def _tc_mid(acc, cnt, r, b, Wl, Wr):
    bm = 1000
    return pl.pallas_call(
        _mid_body,
        grid=(N // bm,),
        in_specs=[pl.BlockSpec((NC, bm, D), lambda i: (0, i, 0)),
                  pl.BlockSpec((NC, bm, D), lambda i: (0, i, 0)),
                  pl.BlockSpec((bm, D), lambda i: (i, 0)),
                  pl.BlockSpec((1, D), lambda i: (0, 0)),
                  pl.BlockSpec((D, D), lambda i: (0, 0)),
                  pl.BlockSpec((D, D), lambda i: (0, 0))],
        out_specs=[pl.BlockSpec((bm, D), lambda i: (i, 0)),
                   pl.BlockSpec((bm, D), lambda i: (i, 0))],
        out_shape=[jax.ShapeDtypeStruct((N, D), jnp.float32),
                   jax.ShapeDtypeStruct((N, D), jnp.float32)],
    )(acc, cnt, r, b.reshape(1, D), Wl, Wr)


def _fin_body(acc_ref, cnt_ref, r_ref, b_ref, w3_ref, b3_ref, o_ref):
    h = _mean_relu(acc_ref, cnt_ref, r_ref, b_ref)
    o_ref[...] = lax.dot_general(h, w3_ref[...], _DOT,
                                 preferred_element_type=jnp.float32) + b3_ref[...]


def _tc_fin(acc, cnt, r, b, W3, b3):
    bm = 1000
    return pl.pallas_call(
        _fin_body,
        grid=(N // bm,),
        in_specs=[pl.BlockSpec((NC, bm, D), lambda i: (0, i, 0)),
                  pl.BlockSpec((NC, bm, D), lambda i: (0, i, 0)),
                  pl.BlockSpec((bm, D), lambda i: (i, 0)),
                  pl.BlockSpec((1, D), lambda i: (0, 0)),
                  pl.BlockSpec((D, D), lambda i: (0, 0)),
                  pl.BlockSpec((1, D), lambda i: (0, 0))],
        out_specs=pl.BlockSpec((bm, D), lambda i: (i, 0)),
        out_shape=jax.ShapeDtypeStruct((N, D), jnp.float32),
    )(acc, cnt, r, b.reshape(1, D), W3, b3.reshape(1, D))


def kernel(x, edge_index, W1l, b1, W1r, W2l, b2, W2r, W3, b3):
    src = edge_index[0]
    dst = edge_index[1]
    pad = EPAD - E
    src_p = jnp.concatenate([src, jnp.zeros((pad,), jnp.int32)]).reshape(NW, CH, C)
    dst_p = jnp.concatenate([dst, jnp.full((pad,), N, jnp.int32)]).reshape(NW, CH, C)
    idxs = jnp.stack([src_p, dst_p], axis=2).reshape(NW, IR, C)
    z128 = jnp.zeros((C, D), jnp.float32)
    ones_in = jnp.ones((C, D), jnp.float32)

    t1, r1 = _tc_pre(x, W1l, W1r)
    acc1, cnt = _sc_aggregate_cnt(t1, idxs, z128, ones_in)
    t2, r2 = _tc_mid(acc1, cnt, r1, b1, W2l, W2r)
    acc2 = _sc_aggregate(t2, idxs, z128)
    return _tc_fin(acc2, cnt, r2, b2, W3, b3)


# speedup vs baseline: 3.5501x; 1.0002x over previous
import jax
import jax.numpy as jnp
from jax import lax
from jax.experimental import pallas as pl
from jax.experimental.pallas import tpu as pltpu
from jax.experimental.pallas import tpu_sc as plsc

N = 10000
E = 320000
D = 128

NC = 2
NS = 16
NW = NC * NS
C = 128
CH = 80
EPAD = NW * C * CH
NP = 10240
RPT = NP // NS
IR = 2 * CH
NG = 2
GCH = CH // NG
GR = 2 * GCH


def _zero_acc(z128, buf_a, acc_sh, r0, sem_a):
    pltpu.sync_copy(z128, buf_a)
    for k in range(RPT // C):
        pltpu.async_copy(buf_a, acc_sh.at[pl.ds(r0 + k * C, C)], sem_a).wait()


def _drain_acc(acc_sh, out, ci, r0, buf_a, sem_a):
    for k in range(RPT // C):
        pltpu.async_copy(acc_sh.at[pl.ds(r0 + k * C, C)], buf_a, sem_a).wait()
        pltpu.sync_copy(buf_a, out.at[ci, pl.ds(r0 + k * C, C)])


def _sc_body_fused(table, idxs, z128, ones_in, acc_out, cnt_out,
                   idx_v, buf_a, buf_b, acc_sh, sem_a, sem_b, sem_s):
    _sc_agg_impl(table, idxs, z128, acc_out, idx_v, buf_a, buf_b, acc_sh,
                 sem_a, sem_b, sem_s)
    ci = lax.axis_index("c")
    si = lax.axis_index("s")
    w = ci * NS + si
    r0 = si * RPT
    plsc.subcore_barrier()
    _zero_acc(z128, buf_a, acc_sh, r0, sem_a)
    pltpu.sync_copy(ones_in, buf_b)
    plsc.subcore_barrier()

    def cgroup(g, carry):
        pltpu.sync_copy(idxs.at[w, pl.ds(g * GR, GR)], idx_v)

        def cbody(t, carry2):
            pltpu.async_copy(buf_b, acc_sh.at[idx_v.at[2 * t + 1]], sem_s,
                             add=True).wait()
            return carry2

        lax.fori_loop(0, GCH, cbody, 0)
        return carry

    lax.fori_loop(0, NG, cgroup, 0)
    plsc.subcore_barrier()
    _drain_acc(acc_sh, cnt_out, ci, r0, buf_a, sem_a)


def _sc_body(table, idxs, z128, acc_out,
             idx_v, buf_a, buf_b, acc_sh, sem_a, sem_b, sem_s):
    _sc_agg_impl(table, idxs, z128, acc_out, idx_v, buf_a, buf_b, acc_sh,
                 sem_a, sem_b, sem_s)


def _sc_agg_impl(table, idxs, z128, acc_out,
                 idx_v, buf_a, buf_b, acc_sh, sem_a, sem_b, sem_s):
    ci = lax.axis_index("c")
    si = lax.axis_index("s")
    w = ci * NS + si
    r0 = si * RPT
    _zero_acc(z128, buf_a, acc_sh, r0, sem_a)
    plsc.subcore_barrier()

    HC = C // 2

    def gat(j, buf, sem):
        pltpu.async_copy(table.at[idx_v.at[2 * j, pl.ds(0, HC)]],
                         buf.at[pl.ds(0, HC)], sem)
        pltpu.async_copy(table.at[idx_v.at[2 * j, pl.ds(HC, HC)]],
                         buf.at[pl.ds(HC, HC)], sem)

    def wat(j, buf, sem):
        pltpu.make_async_copy(table.at[idx_v.at[2 * j, pl.ds(0, HC)]],
                              buf.at[pl.ds(0, HC)], sem).wait()
        pltpu.make_async_copy(table.at[idx_v.at[2 * j, pl.ds(HC, HC)]],
                              buf.at[pl.ds(HC, HC)], sem).wait()

    def group(g, carry):
        pltpu.sync_copy(idxs.at[w, pl.ds(g * GR, GR)], idx_v)
        gat(0, buf_a, sem_a)
        gat(1, buf_b, sem_b)

        def body(tt, carry2):
            j = 2 * tt
            wat(j, buf_a, sem_a)
            pltpu.async_copy(buf_a, acc_sh.at[idx_v.at[2 * j + 1]], sem_s,
                             add=True).wait()

            @pl.when(j + 2 < GCH)
            def _():
                gat(j + 2, buf_a, sem_a)

            wat(j + 1, buf_b, sem_b)
            pltpu.async_copy(buf_b, acc_sh.at[idx_v.at[2 * j + 3]], sem_s,
                             add=True).wait()

            @pl.when(j + 3 < GCH)
            def _():
                gat(j + 3, buf_b, sem_b)

            return carry2

        lax.fori_loop(0, GCH // 2, body, 0)
        return carry

    lax.fori_loop(0, NG, group, 0)
    plsc.subcore_barrier()
    _drain_acc(acc_sh, acc_out, ci, r0, buf_a, sem_a)


def _sc_aggregate(table, idxs, z128):
    mesh = plsc.VectorSubcoreMesh(core_axis_name="c", subcore_axis_name="s")
    kern = pl.kernel(
        _sc_body,
        mesh=mesh,
        out_type=jax.ShapeDtypeStruct((NC, NP, D), jnp.float32),
        scratch_types=[
            pltpu.VMEM((GR, C), jnp.int32),
            pltpu.VMEM((C, D), jnp.float32),
            pltpu.VMEM((C, D), jnp.float32),
            pltpu.VMEM_SHARED((NP, D), jnp.float32),
            pltpu.SemaphoreType.DMA,
            pltpu.SemaphoreType.DMA,
            pltpu.SemaphoreType.DMA,
        ],
    )
    return kern(table, idxs, z128)


def _sc_aggregate_cnt(table, idxs, z128, ones_in):
    mesh = plsc.VectorSubcoreMesh(core_axis_name="c", subcore_axis_name="s")
    kern = pl.kernel(
        _sc_body_fused,
        mesh=mesh,
        out_type=(jax.ShapeDtypeStruct((NC, NP, D), jnp.float32),
                  jax.ShapeDtypeStruct((NC, NP, D), jnp.float32)),
        scratch_types=[
            pltpu.VMEM((GR, C), jnp.int32),
            pltpu.VMEM((C, D), jnp.float32),
            pltpu.VMEM((C, D), jnp.float32),
            pltpu.VMEM_SHARED((NP, D), jnp.float32),
            pltpu.SemaphoreType.DMA,
            pltpu.SemaphoreType.DMA,
            pltpu.SemaphoreType.DMA,
        ],
    )
    return kern(table, idxs, z128, ones_in)


_DOT = (((1,), (1,)), ((), ()))


def _pre_body(x_ref, wl_ref, wr_ref, t_ref, r_ref):
    xb = x_ref[...]
    t_ref[...] = lax.dot_general(xb, wl_ref[...], _DOT,
                                 preferred_element_type=jnp.float32)
    r_ref[...] = lax.dot_general(xb, wr_ref[...], _DOT,
                                 preferred_element_type=jnp.float32)


def _tc_pre(x, W1l, W1r):
    bm = 1000
    return pl.pallas_call(
        _pre_body,
        grid=(N // bm,),
        in_specs=[pl.BlockSpec((bm, D), lambda i: (i, 0)),
                  pl.BlockSpec((D, D), lambda i: (0, 0)),
                  pl.BlockSpec((D, D), lambda i: (0, 0))],
        out_specs=[pl.BlockSpec((bm, D), lambda i: (i, 0)),
                   pl.BlockSpec((bm, D), lambda i: (i, 0))],
        out_shape=[jax.ShapeDtypeStruct((N, D), jnp.float32),
                   jax.ShapeDtypeStruct((N, D), jnp.float32)],
    )(x, W1l, W1r)


def _mean_relu(acc_ref, cnt_ref, r_ref, b_ref):
    acc = acc_ref[0] + acc_ref[1]
    cnt = cnt_ref[0][:, 0:1] + cnt_ref[1][:, 0:1]
    mean = acc / jnp.maximum(cnt, 1.0)
    return jnp.maximum(mean + b_ref[...] + r_ref[...], 0.0)


def _mid_body(acc_ref, cnt_ref, r_ref, b_ref, wl_ref, wr_ref, t_ref, rn_ref):
    h = _mean_relu(acc_ref, cnt_ref, r_ref, b_ref)
    t_ref[...] = lax.dot_general(h, wl_ref[...], _DOT,
                                 preferred_element_type=jnp.float32)
    rn_ref[...] = lax.dot_general(h, wr_ref[...], _DOT,
                                  preferred_element_type=jnp.float32)


def _tc_mid(acc, cnt, r, b, Wl, Wr):
    bm = 1000
    return pl.pallas_call(
        _mid_body,
        grid=(N // bm,),
        in_specs=[pl.BlockSpec((NC, bm, D), lambda i: (0, i, 0)),
                  pl.BlockSpec((NC, bm, D), lambda i: (0, i, 0)),
                  pl.BlockSpec((bm, D), lambda i: (i, 0)),
                  pl.BlockSpec((1, D), lambda i: (0, 0)),
                  pl.BlockSpec((D, D), lambda i: (0, 0)),
                  pl.BlockSpec((D, D), lambda i: (0, 0))],
        out_specs=[pl.BlockSpec((bm, D), lambda i: (i, 0)),
                   pl.BlockSpec((bm, D), lambda i: (i, 0))],
        out_shape=[jax.ShapeDtypeStruct((N, D), jnp.float32),
                   jax.ShapeDtypeStruct((N, D), jnp.float32)],
    )(acc, cnt, r, b.reshape(1, D), Wl, Wr)


def _fin_body(acc_ref, cnt_ref, r_ref, b_ref, w3_ref, b3_ref, o_ref):
    h = _mean_relu(acc_ref, cnt_ref, r_ref, b_ref)
    o_ref[...] = lax.dot_general(h, w3_ref[...], _DOT,
                                 preferred_element_type=jnp.float32) + b3_ref[...]


def _tc_fin(acc, cnt, r, b, W3, b3):
    bm = 1000
    return pl.pallas_call(
        _fin_body,
        grid=(N // bm,),
        in_specs=[pl.BlockSpec((NC, bm, D), lambda i: (0, i, 0)),
                  pl.BlockSpec((NC, bm, D), lambda i: (0, i, 0)),
                  pl.BlockSpec((bm, D), lambda i: (i, 0)),
                  pl.BlockSpec((1, D), lambda i: (0, 0)),
                  pl.BlockSpec((D, D), lambda i: (0, 0)),
                  pl.BlockSpec((1, D), lambda i: (0, 0))],
        out_specs=pl.BlockSpec((bm, D), lambda i: (i, 0)),
        out_shape=jax.ShapeDtypeStruct((N, D), jnp.float32),
    )(acc, cnt, r, b.reshape(1, D), W3, b3.reshape(1, D))


def kernel(x, edge_index, W1l, b1, W1r, W2l, b2, W2r, W3, b3):
    src = edge_index[0]
    dst = edge_index[1]
    pad = EPAD - E
    src_p = jnp.concatenate([src, jnp.zeros((pad,), jnp.int32)]).reshape(NW, CH, C)
    dst_p = jnp.concatenate([dst, jnp.full((pad,), N, jnp.int32)]).reshape(NW, CH, C)
    idxs = jnp.stack([src_p, dst_p], axis=2).reshape(NW, IR, C)
    z128 = jnp.zeros((C, D), jnp.float32)
    ones_in = jnp.ones((C, D), jnp.float32)

    t1, r1 = _tc_pre(x, W1l, W1r)
    acc1, cnt = _sc_aggregate_cnt(t1, idxs, z128, ones_in)
    t2, r2 = _tc_mid(acc1, cnt, r1, b1, W2l, W2r)
    acc2 = _sc_aggregate(t2, idxs, z128)
    return _tc_fin(acc2, cnt, r2, b2, W3, b3)
